# baseline (speedup 1.0000x reference)
import functools

import jax
import jax.numpy as jnp
from jax import lax
from jax.experimental import pallas as pl
from jax.experimental.pallas import tpu as pltpu
from jax.experimental.pallas import tpu_sc as plsc

VOCAB = 1000000
HIDDEN = 64
BATCH = 4096
SEQ = 200

NC = 2
NS = 16
NW = NC * NS

B = BATCH * SEQ
B_PER_W = B // NW
C = 256
N_CHUNKS = B_PER_W // C
GPB = 1
BIG = GPB * C
NBIG = 5
N_BIG = B_PER_W // BIG
N_ROUNDS = N_BIG // NBIG

assert B_PER_W * NW == B and BIG * N_BIG == B_PER_W and NBIG * N_ROUNDS == N_BIG

_mesh = plsc.VectorSubcoreMesh(core_axis_name="c", subcore_axis_name="s")


@functools.partial(
    pl.kernel,
    out_type=jax.ShapeDtypeStruct((B, HIDDEN), jnp.bfloat16),
    mesh=_mesh,
    scratch_types=[
        pltpu.VMEM((N_CHUNKS, C), jnp.int32),
        [pltpu.VMEM((BIG, HIDDEN), jnp.bfloat16) for _ in range(NBIG)],
        [pltpu.SemaphoreType.DMA for _ in range(NBIG)],
        [pltpu.SemaphoreType.DMA for _ in range(NBIG)],
    ],
    compiler_params=pltpu.CompilerParams(use_tc_tiling_on_sc=False),
)
def _emb_gather(table_hbm, idx_hbm, out_hbm, idx_v, bufs, sem_g, sem_s):
    wid = lax.axis_index("s") * NC + lax.axis_index("c")
    base = wid * B_PER_W

    pltpu.sync_copy(idx_hbm.at[wid], idx_v)

    def start_big_gather(k, i):
        for q in range(GPB):
            pltpu.make_async_copy(
                table_hbm.at[idx_v.at[k * GPB + q]],
                bufs[i].at[pl.ds(q * C, C)],
                sem_g[i],
            ).start()

    def wait_big_gather(i):
        for q in range(GPB):
            pltpu.make_async_copy(
                table_hbm.at[idx_v.at[0]],
                bufs[i].at[pl.ds(q * C, C)],
                sem_g[i],
            ).wait()

    def start_scatter(k, i):
        pltpu.make_async_copy(
            bufs[i], out_hbm.at[pl.ds(base + k * BIG, BIG)], sem_s[i]
        ).start()

    def wait_scatter(i):
        pltpu.make_async_copy(
            bufs[i], out_hbm.at[pl.ds(base, BIG)], sem_s[i]
        ).wait()

    for i in range(NBIG):
        start_big_gather(i, i)

    def round_body(r, _):
        for i in range(NBIG):
            k = r * NBIG + i
            wait_big_gather(i)
            start_scatter(k, i)
            wait_scatter(i)
            start_big_gather(k + NBIG, i)
        return _

    lax.fori_loop(0, N_ROUNDS - 1, round_body, 0, unroll=False)

    for i in range(NBIG):
        k = (N_ROUNDS - 1) * NBIG + i
        wait_big_gather(i)
        start_scatter(k, i)
    for i in range(NBIG):
        wait_scatter(i)


def kernel(input_ids, emb):
    ids = input_ids.reshape(-1).astype(jnp.int32)
    idx3 = ids.reshape(NW, N_CHUNKS, C)
    out = _emb_gather(emb.astype(jnp.bfloat16), idx3)
    return out.astype(jnp.float32).reshape(BATCH, SEQ, HIDDEN)


# speedup vs baseline: 1.4463x; 1.4463x over previous
import functools

import jax
import jax.numpy as jnp
from jax import lax
from jax.experimental import pallas as pl
from jax.experimental.pallas import tpu as pltpu
from jax.experimental.pallas import tpu_sc as plsc

VOCAB = 1000000
HIDDEN = 64
BATCH = 4096
SEQ = 200

NC = 2
NS = 16
NW = NC * NS

B = BATCH * SEQ
B_PER_W = B // NW
C = 256
N_CHUNKS = B_PER_W // C
GPB = 1
BIG = GPB * C
NBIG = 5
N_BIG = B_PER_W // BIG
N_ROUNDS = N_BIG // NBIG

assert B_PER_W * NW == B and BIG * N_BIG == B_PER_W and NBIG * N_ROUNDS == N_BIG

_mesh = plsc.VectorSubcoreMesh(core_axis_name="c", subcore_axis_name="s")


@functools.partial(
    pl.kernel,
    out_type=jax.ShapeDtypeStruct((B, HIDDEN), jnp.float32),
    mesh=_mesh,
    scratch_types=[
        pltpu.VMEM((N_CHUNKS, C), jnp.int32),
        [pltpu.VMEM((BIG, HIDDEN), jnp.float32) for _ in range(NBIG)],
        [pltpu.SemaphoreType.DMA for _ in range(NBIG)],
        [pltpu.SemaphoreType.DMA for _ in range(NBIG)],
    ],
    compiler_params=pltpu.CompilerParams(use_tc_tiling_on_sc=False),
)
def _emb_gather(table_hbm, idx_hbm, out_hbm, idx_v, bufs, sem_g, sem_s):
    wid = lax.axis_index("s") * NC + lax.axis_index("c")
    base = wid * B_PER_W

    pltpu.sync_copy(idx_hbm.at[wid], idx_v)

    def start_big_gather(k, i):
        for q in range(GPB):
            pltpu.make_async_copy(
                table_hbm.at[idx_v.at[k * GPB + q]],
                bufs[i].at[pl.ds(q * C, C)],
                sem_g[i],
            ).start()

    def wait_big_gather(i):
        for q in range(GPB):
            pltpu.make_async_copy(
                table_hbm.at[idx_v.at[0]],
                bufs[i].at[pl.ds(q * C, C)],
                sem_g[i],
            ).wait()

    def start_scatter(k, i):
        pltpu.make_async_copy(
            bufs[i], out_hbm.at[pl.ds(base + k * BIG, BIG)], sem_s[i]
        ).start()

    def wait_scatter(i):
        pltpu.make_async_copy(
            bufs[i], out_hbm.at[pl.ds(base, BIG)], sem_s[i]
        ).wait()

    for i in range(NBIG):
        start_big_gather(i, i)

    def round_body(r, _):
        for i in range(NBIG):
            k = r * NBIG + i
            wait_big_gather(i)
            start_scatter(k, i)
            wait_scatter(i)
            start_big_gather(k + NBIG, i)
        return _

    lax.fori_loop(0, N_ROUNDS - 1, round_body, 0, unroll=False)

    for i in range(NBIG):
        k = (N_ROUNDS - 1) * NBIG + i
        wait_big_gather(i)
        start_scatter(k, i)
    for i in range(NBIG):
        wait_scatter(i)


def kernel(input_ids, emb):
    ids = input_ids.reshape(-1).astype(jnp.int32)
    idx3 = ids.reshape(NW, N_CHUNKS, C)
    out = _emb_gather(emb, idx3)
    return out.reshape(BATCH, SEQ, HIDDEN)
